# baseline (speedup 1.0000x reference)
import jax
import jax.numpy as jnp
from jax import lax
from jax.experimental import pallas as pl
from jax.experimental.pallas import tpu as pltpu
from jax.experimental.pallas import tpu_sc as plsc

B, L, D = 1024, 20, 64
HOURS = 24 * 7
NC, NS = 2, 16
NW = NC * NS
ROWS = B * L
RPW = ROWS // NW
GCHUNK = 128
NCHUNK = RPW // GCHUNK


def _sc_joint_body(wt_hbm, wl_hbm, wu_hbm, traj_hbm, out_hbm,
                   traj_v, uidx_v, lidx_v, tidx_v, au_v, al_v, at_v, sem):
    wid = lax.axis_index("s") * NC + lax.axis_index("c")
    base = wid * RPW
    pltpu.sync_copy(traj_hbm.at[pl.ds(base * 3, RPW * 3)], traj_v)

    def split(k, _):
        off = 48 * k + 3 * lax.iota(jnp.int32, 16)
        u = plsc.load_gather(traj_v, [off])
        l = plsc.load_gather(traj_v, [off + 1])
        t = plsc.load_gather(traj_v, [off + 2])
        sl = pl.ds(k * 16, 16)
        uidx_v[sl] = u
        lidx_v[sl] = l
        tidx_v[sl] = (t + (HOURS - 1)) % HOURS + 1
        return 0

    lax.fori_loop(0, RPW // 16, split, 0, unroll=4)

    cps = []
    for idx_v, table, dst_v in ((uidx_v, wu_hbm, au_v),
                                (lidx_v, wl_hbm, al_v),
                                (tidx_v, wt_hbm, at_v)):
        for k in range(NCHUNK):
            cps.append(pltpu.async_copy(
                table.at[idx_v.at[pl.ds(k * GCHUNK, GCHUNK)]],
                dst_v.at[pl.ds(k * GCHUNK, GCHUNK), :], sem))
    for cp in cps:
        cp.wait()

    def addrow(r, _):
        for c in range(D // 16):
            sl = pl.ds(c * 16, 16)
            au_v[r, sl] = au_v[r, sl] + al_v[r, sl] + at_v[r, sl]
        return 0

    lax.fori_loop(0, RPW, addrow, 0, unroll=4)
    pltpu.sync_copy(au_v, out_hbm.at[pl.ds(base, RPW)])


def _sc_joint(W_t, W_l, W_u, traj_flat):
    mesh = plsc.VectorSubcoreMesh(core_axis_name="c", subcore_axis_name="s")
    return pl.kernel(
        _sc_joint_body,
        out_type=jax.ShapeDtypeStruct((ROWS, D), jnp.float32),
        mesh=mesh,
        scratch_types=[
            pltpu.VMEM((RPW * 3,), jnp.int32),
            pltpu.VMEM((RPW,), jnp.int32),
            pltpu.VMEM((RPW,), jnp.int32),
            pltpu.VMEM((RPW,), jnp.int32),
            pltpu.VMEM((RPW, D), jnp.float32),
            pltpu.VMEM((RPW, D), jnp.float32),
            pltpu.VMEM((RPW, D), jnp.float32),
            pltpu.SemaphoreType.DMA,
        ],
        compiler_params=pltpu.CompilerParams(use_tc_tiling_on_sc=False,
                                             needs_layout_passes=False),
    )(W_t, W_l, W_u, traj_flat)


BB = 32


def _tc_delta_body(len_ref, mat_ref, wsu_ref, wsl_ref, wtu_ref,
                   wtl_ref, out_ref):
    wsl = wsl_ref[:, :]
    wsu = wsu_ref[:, :]
    wtl = wtl_ref[:, :]
    wtu = wtu_ref[:, :]
    basev = wsl + wtl
    svec = wsu - wsl
    tvec = wtu - wtl
    dbase = basev[1] - basev[0]
    dsv = svec[1] - svec[0]
    dtv = tvec[1] - tvec[0]

    x = mat_ref[:, :].reshape(BB, L * L, 2)
    ds = x[:, :, 0]
    dt = x[:, :, 1]

    p = lax.broadcasted_iota(jnp.int32, (BB, L * L), 1)
    i = p // L
    j = p - i * L
    ln = len_ref[:, :]
    m = ((i < ln) & (j < ln)).astype(jnp.float32)[:, :, None]

    ds = ds[:, :, None]
    dt = dt[:, :, None]
    u = basev[0] + ds * svec[0] + dt * tvec[0]
    v = dbase + ds * dsv + dt * dtv
    out_ref[:, :, :] = u + m * v


def _tc_delta(traj_length2d, mat2, W_su, W_sl, W_tu, W_tl):
    grid = (B // BB,)
    return pl.pallas_call(
        _tc_delta_body,
        grid=grid,
        in_specs=[
            pl.BlockSpec((BB, 1), lambda b: (b, 0)),
            pl.BlockSpec((BB, 2 * L * L), lambda b: (b, 0)),
            pl.BlockSpec((2, D), lambda b: (0, 0)),
            pl.BlockSpec((2, D), lambda b: (0, 0)),
            pl.BlockSpec((2, D), lambda b: (0, 0)),
            pl.BlockSpec((2, D), lambda b: (0, 0)),
        ],
        out_specs=pl.BlockSpec((BB, L * L, D), lambda b: (b, 0, 0)),
        out_shape=jax.ShapeDtypeStruct((B, L * L, D), jnp.float32),
        compiler_params=pltpu.CompilerParams(
            dimension_semantics=("arbitrary",)),
    )(traj_length2d, mat2, W_su, W_sl, W_tu, W_tl)


def kernel(traj_input, mat_input, traj_length, W_t, W_l, W_u, W_su, W_sl,
           W_tu, W_tl):
    traj_flat = traj_input.reshape(ROWS * 3)
    joint = _sc_joint(W_t, W_l, W_u, traj_flat).reshape(B, L, D)

    mat2 = mat_input.reshape(B, 2 * L * L)
    delta = _tc_delta(traj_length.reshape(B, 1), mat2, W_su, W_sl, W_tu,
                      W_tl).reshape(B, L, L, D)
    return (joint, delta)
